# baseline (speedup 1.0000x reference)
import jax
import jax.numpy as jnp
from jax.experimental import pallas as pl
from jax.experimental.pallas import tpu as pltpu

_B = 1024
_K = 100000
_L = 64
_KB = 2048
_NSTEPS = (_K + _KB - 1) // _KB


def _mf_kernel(u_ref, i_ref, wu_ref, wi_ref, bias_ref, gb_ref, out_ref,
               acc_u, acc_i, acc_b):
    step = pl.program_id(0)

    @pl.when(step == 0)
    def _init():
        acc_u[...] = jnp.zeros_like(acc_u)
        acc_i[...] = jnp.zeros_like(acc_i)
        acc_b[...] = jnp.zeros_like(acc_b)

    row = jax.lax.broadcasted_iota(jnp.int32, (_KB, 1), 0)
    valid = (step * _KB + row) < _K
    wu = jnp.where(valid, wu_ref[...], 0.0)
    wi = jnp.where(valid, wi_ref[...], 0.0)
    bias = jnp.where(valid, bias_ref[...], 0.0)

    col = jax.lax.broadcasted_iota(jnp.int32, (1, _KB), 1)
    valid_c = (step * _KB + col) < _K
    u = jnp.where(valid_c, u_ref[...], 0.0)
    it = jnp.where(valid_c, i_ref[...], 0.0)
    acc_u[...] += jnp.dot(u, wu, preferred_element_type=jnp.float32)
    acc_i[...] += jnp.dot(it, wi, preferred_element_type=jnp.float32)
    acc_b[...] += jnp.dot(it, bias, preferred_element_type=jnp.float32)

    @pl.when(step == _NSTEPS - 1)
    def _finalize():
        pred = jnp.sum(acc_u[...] * acc_i[...], axis=1, keepdims=True)
        out_ref[...] = pred + acc_b[...] + gb_ref[0]


def kernel(user_features, item_features, user_latent_weight,
           item_latent_weight, item_biases_weight, global_bias):
    out = pl.pallas_call(
        _mf_kernel,
        grid=(_NSTEPS,),
        in_specs=[
            pl.BlockSpec((_B, _KB), lambda k: (0, k)),
            pl.BlockSpec((_B, _KB), lambda k: (0, k)),
            pl.BlockSpec((_KB, _L), lambda k: (k, 0)),
            pl.BlockSpec((_KB, _L), lambda k: (k, 0)),
            pl.BlockSpec((_KB, 1), lambda k: (k, 0)),
            pl.BlockSpec(memory_space=pltpu.SMEM),
        ],
        out_specs=pl.BlockSpec((_B, 1), lambda k: (0, 0)),
        out_shape=jax.ShapeDtypeStruct((_B, 1), jnp.float32),
        scratch_shapes=[
            pltpu.VMEM((_B, _L), jnp.float32),
            pltpu.VMEM((_B, _L), jnp.float32),
            pltpu.VMEM((_B, 1), jnp.float32),
        ],
        compiler_params=pltpu.CompilerParams(
            dimension_semantics=("arbitrary",),
        ),
    )(user_features, item_features, user_latent_weight,
      item_latent_weight, item_biases_weight, global_bias)
    return out.reshape(_B)
